# baseline (device time: 34348 ns/iter reference)
import jax
import jax.numpy as jnp
from jax import lax
from jax.experimental import pallas as pl
from jax.experimental.pallas import tpu as pltpu

N_DEV = 8
N_LAYERS = 3


def kernel(
    x,
    Win0,
    Wout0,
    Win1,
    Wout1,
    Win2,
    Wout2,
):
    b, d = x.shape
    ck = d // N_DEV
    crows, ccols = b // 2, 2 * ck

    def body(
        x_ref,
        win0_ref,
        wout0_ref,
        win1_ref,
        wout1_ref,
        win2_ref,
        wout2_ref,
        out_ref,
        own_ref,
        rs_ref,
        agown_ref,
        ag_ref,
        rs_send,
        rs_recv,
        ag_send,
        ag_recv,
    ):
        my = lax.axis_index("i")
        wins = [win0_ref, win1_ref, win2_ref]
        wouts = [wout0_ref, wout1_ref, wout2_ref]

        barrier = pltpu.get_barrier_semaphore()

        xb = x_ref[...]
        for l in range(N_LAYERS):
            h = jnp.dot(xb, wins[l][...], preferred_element_type=jnp.float32)
            h = jnp.maximum(h, 0.0).astype(jnp.bfloat16)
            partial = jnp.dot(
                h, wouts[l][...], preferred_element_type=jnp.float32
            )
            for j in range(N_DEV):
                r, c = divmod(j, 4)
                own_ref[l, j] = partial[
                    r * crows : (r + 1) * crows,
                    c * ccols : (c + 1) * ccols,
                ].astype(jnp.bfloat16)

            if l == 0:
                for off in range(1, N_DEV):
                    peer = lax.rem(my + off, N_DEV)
                    pl.semaphore_signal(
                        barrier,
                        inc=1,
                        device_id=(peer,),
                        device_id_type=pl.DeviceIdType.MESH,
                    )
                pl.semaphore_wait(barrier, N_DEV - 1)

            rs_rdmas = []
            for off in range(1, N_DEV):
                peer = lax.rem(my + off, N_DEV)
                slot = (N_DEV - off) - 1
                rdma = pltpu.make_async_remote_copy(
                    src_ref=own_ref.at[l, peer],
                    dst_ref=rs_ref.at[l, slot],
                    send_sem=rs_send.at[l, off - 1],
                    recv_sem=rs_recv.at[l, slot],
                    device_id=(peer,),
                    device_id_type=pl.DeviceIdType.MESH,
                )
                rdma.start()
                rs_rdmas.append(rdma)

            acc = own_ref[l, my].astype(jnp.float32)
            for o in range(1, N_DEV):
                src = lax.rem(my + o, N_DEV)
                recv = pltpu.make_async_remote_copy(
                    src_ref=own_ref.at[l, my],
                    dst_ref=rs_ref.at[l, o - 1],
                    send_sem=rs_send.at[l, o - 1],
                    recv_sem=rs_recv.at[l, o - 1],
                    device_id=(src,),
                    device_id_type=pl.DeviceIdType.MESH,
                )
                recv.wait_recv()
                acc = acc + rs_ref[l, o - 1].astype(jnp.float32)

            accb = acc.astype(jnp.bfloat16)
            agown_ref[l] = accb
            ag_ref[l, my] = accb

            ag_rdmas = []
            for off in range(1, N_DEV):
                peer = lax.rem(my + off, N_DEV)
                slot = (N_DEV - off) - 1
                rdma = pltpu.make_async_remote_copy(
                    src_ref=agown_ref.at[l],
                    dst_ref=ag_ref.at[l, my],
                    send_sem=ag_send.at[l, off - 1],
                    recv_sem=ag_recv.at[l, slot],
                    device_id=(peer,),
                    device_id_type=pl.DeviceIdType.MESH,
                )
                rdma.start()
                ag_rdmas.append(rdma)

            for o in range(1, N_DEV):
                src = lax.rem(my + o, N_DEV)
                recv = pltpu.make_async_remote_copy(
                    src_ref=agown_ref.at[l],
                    dst_ref=ag_ref.at[l, src],
                    send_sem=ag_send.at[l, o - 1],
                    recv_sem=ag_recv.at[l, o - 1],
                    device_id=(src,),
                    device_id_type=pl.DeviceIdType.MESH,
                )
                recv.wait_recv()

            nxt = jnp.concatenate(
                [
                    jnp.concatenate(
                        [ag_ref[l, 4 * r + c] for c in range(4)], axis=1
                    )
                    for r in range(2)
                ],
                axis=0,
            )

            for rdma in rs_rdmas:
                rdma.wait_send()
            for rdma in ag_rdmas:
                rdma.wait_send()

            if l < N_LAYERS - 1:
                xb = nxt
            else:
                out_ref[...] = nxt

    return pl.pallas_call(
        body,
        out_shape=jax.ShapeDtypeStruct((b, d), jnp.bfloat16),
        in_specs=[pl.BlockSpec(memory_space=pltpu.VMEM)] * 7,
        out_specs=pl.BlockSpec(memory_space=pltpu.VMEM),
        scratch_shapes=[
            pltpu.VMEM((N_LAYERS, N_DEV, crows, ccols), jnp.bfloat16),
            pltpu.VMEM((N_LAYERS, N_DEV - 1, crows, ccols), jnp.bfloat16),
            pltpu.VMEM((N_LAYERS, crows, ccols), jnp.bfloat16),
            pltpu.VMEM((N_LAYERS, N_DEV, crows, ccols), jnp.bfloat16),
            pltpu.SemaphoreType.DMA((N_LAYERS, N_DEV - 1)),
            pltpu.SemaphoreType.DMA((N_LAYERS, N_DEV - 1)),
            pltpu.SemaphoreType.DMA((N_LAYERS, N_DEV - 1)),
            pltpu.SemaphoreType.DMA((N_LAYERS, N_DEV - 1)),
        ],
        compiler_params=pltpu.CompilerParams(collective_id=0),
    )(
        x.astype(jnp.bfloat16),
        Win0.astype(jnp.bfloat16),
        Wout0.astype(jnp.bfloat16),
        Win1.astype(jnp.bfloat16),
        Wout1.astype(jnp.bfloat16),
        Win2.astype(jnp.bfloat16),
        Wout2.astype(jnp.bfloat16),
    )


# device time: 31728 ns/iter; 1.0826x vs baseline; 1.0826x over previous
import jax
import jax.numpy as jnp
from jax import lax
from jax.experimental import pallas as pl
from jax.experimental.pallas import tpu as pltpu

N_DEV = 8
N_LAYERS = 3


def kernel(
    x,
    Win0,
    Wout0,
    Win1,
    Wout1,
    Win2,
    Wout2,
):
    b, d = x.shape

    def body(
        x_ref,
        win0_ref,
        wout0_ref,
        win1_ref,
        wout1_ref,
        win2_ref,
        wout2_ref,
        out_ref,
        comm_ref,
        own_ref,
        send_sems,
        recv_sems,
    ):
        my = lax.axis_index("i")
        wins = [win0_ref, win1_ref, win2_ref]
        wouts = [wout0_ref, wout1_ref, wout2_ref]

        barrier = pltpu.get_barrier_semaphore()

        xb = x_ref[...]
        all_sends = []
        for l in range(N_LAYERS):
            h = jnp.dot(xb, wins[l][...], preferred_element_type=jnp.float32)
            h = jnp.maximum(h, 0.0).astype(jnp.bfloat16)
            partial = jnp.dot(
                h, wouts[l][...], preferred_element_type=jnp.float32
            )
            own_ref[l] = partial.astype(jnp.bfloat16)

            if l == 0:
                for off in range(1, N_DEV):
                    peer = lax.rem(my + off, N_DEV)
                    pl.semaphore_signal(
                        barrier,
                        inc=1,
                        device_id=(peer,),
                        device_id_type=pl.DeviceIdType.MESH,
                    )
                pl.semaphore_wait(barrier, N_DEV - 1)

            sends = []
            for off in range(1, N_DEV):
                peer = lax.rem(my + off, N_DEV)
                slot = (N_DEV - off) - 1
                rdma = pltpu.make_async_remote_copy(
                    src_ref=own_ref.at[l],
                    dst_ref=comm_ref.at[l, slot],
                    send_sem=send_sems.at[l, off - 1],
                    recv_sem=recv_sems.at[l, slot],
                    device_id=(peer,),
                    device_id_type=pl.DeviceIdType.MESH,
                )
                rdma.start()
                sends.append(rdma)

            acc = partial
            for o in range(1, N_DEV):
                src = lax.rem(my + o, N_DEV)
                recv = pltpu.make_async_remote_copy(
                    src_ref=own_ref.at[l],
                    dst_ref=comm_ref.at[l, o - 1],
                    send_sem=send_sems.at[l, o - 1],
                    recv_sem=recv_sems.at[l, o - 1],
                    device_id=(src,),
                    device_id_type=pl.DeviceIdType.MESH,
                )
                recv.wait_recv()
                acc = acc + comm_ref[l, o - 1].astype(jnp.float32)

            all_sends.extend(sends)

            if l < N_LAYERS - 1:
                xb = acc.astype(jnp.bfloat16)
            else:
                out_ref[...] = acc.astype(jnp.bfloat16)

        for rdma in all_sends:
            rdma.wait_send()

    return pl.pallas_call(
        body,
        out_shape=jax.ShapeDtypeStruct((b, d), jnp.bfloat16),
        in_specs=[pl.BlockSpec(memory_space=pltpu.VMEM)] * 7,
        out_specs=pl.BlockSpec(memory_space=pltpu.VMEM),
        scratch_shapes=[
            pltpu.VMEM((N_LAYERS, N_DEV - 1, b, d), jnp.bfloat16),
            pltpu.VMEM((N_LAYERS, b, d), jnp.bfloat16),
            pltpu.SemaphoreType.DMA((N_LAYERS, N_DEV - 1)),
            pltpu.SemaphoreType.DMA((N_LAYERS, N_DEV - 1)),
        ],
        compiler_params=pltpu.CompilerParams(collective_id=0),
    )(
        x.astype(jnp.bfloat16),
        Win0.astype(jnp.bfloat16),
        Wout0.astype(jnp.bfloat16),
        Win1.astype(jnp.bfloat16),
        Wout1.astype(jnp.bfloat16),
        Win2.astype(jnp.bfloat16),
        Wout2.astype(jnp.bfloat16),
    )
